# baseline (speedup 1.0000x reference)
import functools

import jax
import jax.numpy as jnp
from jax import lax
from jax.experimental import pallas as pl
from jax.experimental.pallas import tpu as pltpu
from jax.experimental.pallas import tpu_sc as plsc

NC = 2
NS = 16
NT = NC * NS
CHUNK = 128
VEC = 16
W_WIN = 8192
RANGE = 320
CL = 7680


def _sc_mesh():
    return plsc.VectorSubcoreMesh(core_axis_name="c", subcore_axis_name="s")


def _build_prep(N, R, E_pad):
    n_win = E_pad // W_WIN
    garb0 = NT * RANGE
    z0 = R * N

    def body(src_hbm, typ_hbm, dst_hbm, gl_hbm, dl_hbm, deg_hbm,
             wsrc, wtyp, wdst, pbufg, pbufd, hist, tgl, tdl, degf):
        c = lax.axis_index("c")
        s = lax.axis_index("s")
        t = s * NC + c
        lo = t * RANGE
        hi = lo + RANGE
        lane = lax.iota(jnp.int32, VEC)

        def fill_tmpl(i, _):
            iv = i * VEC + lane
            tgl[pl.ds(i * VEC, VEC)] = z0 + (iv % 128)
            tdl[pl.ds(i * VEC, VEC)] = jnp.broadcast_to(garb0, (VEC,)
                                                        ).astype(jnp.int32)
            return 0
        lax.fori_loop(0, 1280 // VEC, fill_tmpl, 0)
        def zero_hist(i, _):
            hist[pl.ds(i * VEC, VEC)] = jnp.zeros((VEC,), jnp.int32)
            return 0
        lax.fori_loop(0, RANGE // VEC, zero_hist, 0)

        def pre_body(k, _):
            off = pl.multiple_of(t * CL + k * 1280, 1280)
            pltpu.sync_copy(tgl, gl_hbm.at[pl.ds(off, 1280)])
            pltpu.sync_copy(tdl, dl_hbm.at[pl.ds(off, 1280)])
            return 0
        lax.fori_loop(0, CL // 1280, pre_body, 0)

        def win_body(win, flushed):
            wb = pl.multiple_of(win * W_WIN, W_WIN)
            pltpu.sync_copy(src_hbm.at[pl.ds(wb, W_WIN)], wsrc)
            pltpu.sync_copy(typ_hbm.at[pl.ds(wb, W_WIN)], wtyp)
            pltpu.sync_copy(dst_hbm.at[pl.ds(wb, W_WIN)], wdst)

            def grp_body(g, cnt):
                sl = pl.ds(g * VEC, VEC)
                dv = wdst[sl]
                own = (dv >= lo) & (dv < hi)
                gv = wtyp[sl] * N + wsrc[sl]
                ec = jnp.zeros((VEC,), jnp.int32)
                lc = jnp.zeros((VEC,), jnp.int32)
                for k in range(1, VEC):
                    re_idx = (lane - k) % VEC
                    rl_idx = (lane + k) % VEC
                    eqe = (dv == dv[re_idx]) & (lane >= k)
                    eql = (dv == dv[rl_idx]) & (lane < VEC - k)
                    ec = ec + eqe.astype(jnp.int32)
                    lc = lc + eql.astype(jnp.int32)
                dloc = jnp.where(own, dv - lo, 0)
                hbase = plsc.load_gather(hist, [dloc])
                is_last = own & (lc == 0)
                plsc.store_scatter(hist, [dloc], hbase + ec + lc + 1,
                                   mask=is_last)
                pref = plsc.cumsum(own.astype(jnp.int32))
                pos = cnt + pref - 1
                plsc.store_scatter(pbufg, [pos], gv, mask=own)
                plsc.store_scatter(pbufd, [pos], dv, mask=own)
                return cnt + jnp.max(pref)
            cnt = lax.fori_loop(0, W_WIN // VEC, grp_body, 0)

            for k in range(CHUNK // VEC):
                pos = cnt + k * VEC + lane
                plsc.store_scatter(pbufg, [pos], z0 + ((lane + k * VEC) % 128))
                plsc.store_scatter(pbufd, [pos],
                                   jnp.broadcast_to(garb0, (VEC,)
                                                    ).astype(jnp.int32))
            cntp = ((cnt + CHUNK - 1) // CHUNK) * CHUNK

            nfl = jnp.minimum(cntp, CL - flushed) // CHUNK

            def flush_body(k, _):
                src_sl = pl.ds(k * CHUNK, CHUNK)
                dst_off = pl.multiple_of(t * CL + flushed + k * CHUNK, CHUNK)
                pltpu.sync_copy(pbufg.at[src_sl],
                                gl_hbm.at[pl.ds(dst_off, CHUNK)])
                pltpu.sync_copy(pbufd.at[src_sl],
                                dl_hbm.at[pl.ds(dst_off, CHUNK)])
                return 0
            lax.fori_loop(0, nfl, flush_body, 0)
            return flushed + nfl * CHUNK

        lax.fori_loop(0, n_win, win_body, 0)

        def deg_cvt(i, _):
            degf[pl.ds(i * VEC, VEC)] = hist[pl.ds(i * VEC, VEC)].astype(
                jnp.float32)
            return 0
        lax.fori_loop(0, RANGE // VEC, deg_cvt, 0)
        pltpu.sync_copy(degf, deg_hbm.at[pl.ds(pl.multiple_of(lo, RANGE),
                                               RANGE)])

    return pl.kernel(
        body,
        out_type=(jax.ShapeDtypeStruct((NT * CL,), jnp.int32),
                  jax.ShapeDtypeStruct((NT * CL,), jnp.int32),
                  jax.ShapeDtypeStruct((NT * RANGE,), jnp.float32)),
        mesh=_sc_mesh(),
        scratch_types=[
            pltpu.VMEM((W_WIN,), jnp.int32),
            pltpu.VMEM((W_WIN,), jnp.int32),
            pltpu.VMEM((W_WIN,), jnp.int32),
            pltpu.VMEM((W_WIN + CHUNK,), jnp.int32),
            pltpu.VMEM((W_WIN + CHUNK,), jnp.int32),
            pltpu.VMEM((RANGE,), jnp.int32),
            pltpu.VMEM((1280,), jnp.int32),
            pltpu.VMEM((1280,), jnp.int32),
            pltpu.VMEM((RANGE,), jnp.float32),
        ],
        compiler_params=pltpu.CompilerParams(needs_layout_passes=False),
    )


def _build_gather_m(D):
    def body(xr_hbm, gl_hbm, m_hbm, gbuf, rows_v, sem):
        c = lax.axis_index("c")
        s = lax.axis_index("s")
        t = s * NC + c

        def chunk_body(k, _):
            base = pl.multiple_of(t * CL + k * CHUNK, CHUNK)
            pltpu.sync_copy(gl_hbm.at[pl.ds(base, CHUNK)], gbuf)
            pltpu.async_copy(xr_hbm.at[gbuf], rows_v, sem).wait()
            pltpu.sync_copy(rows_v, m_hbm.at[pl.ds(base, CHUNK)])
            return 0
        lax.fori_loop(0, CL // CHUNK, chunk_body, 0)

    return pl.kernel(
        body,
        out_type=jax.ShapeDtypeStruct((NT * CL, D), jnp.float32),
        mesh=_sc_mesh(),
        scratch_types=[
            pltpu.VMEM((CHUNK,), jnp.int32),
            pltpu.VMEM((CHUNK, D), jnp.float32),
            pltpu.SemaphoreType.DMA,
        ],
    )


def _build_pair_gather(N, D, B):
    per_w = B // NT

    def body(x_hbm, heads_hbm, tails_hbm, zh_hbm, zt_hbm, ibuf, rows_v, sem):
        c = lax.axis_index("c")
        s = lax.axis_index("s")
        w = s * NC + c
        base = pl.multiple_of(w * per_w, per_w)
        pltpu.sync_copy(heads_hbm.at[pl.ds(base, per_w)], ibuf)
        pltpu.async_copy(x_hbm.at[ibuf], rows_v, sem).wait()
        pltpu.sync_copy(rows_v, zh_hbm.at[pl.ds(base, per_w)])
        pltpu.sync_copy(tails_hbm.at[pl.ds(base, per_w)], ibuf)
        pltpu.async_copy(x_hbm.at[ibuf], rows_v, sem).wait()
        pltpu.sync_copy(rows_v, zt_hbm.at[pl.ds(base, per_w)])

    return pl.kernel(
        body,
        out_type=(jax.ShapeDtypeStruct((B, D), jnp.float32),
                  jax.ShapeDtypeStruct((B, D), jnp.float32)),
        mesh=_sc_mesh(),
        scratch_types=[
            pltpu.VMEM((per_w,), jnp.int32),
            pltpu.VMEM((per_w, D), jnp.float32),
            pltpu.SemaphoreType.DMA,
        ],
    )


def _xr_body(nb, x_ref, w_ref, out_ref):
    j = pl.program_id(0)

    @pl.when(j < nb)
    def _():
        out_ref[...] = jnp.dot(x_ref[...], w_ref[0],
                               preferred_element_type=jnp.float32)

    @pl.when(j >= nb)
    def _():
        out_ref[...] = jnp.zeros_like(out_ref)


def _relation_transform(x, w_l, bn):
    N, D = x.shape
    R = w_l.shape[0]
    npb = N // bn
    nb = R * npb
    return pl.pallas_call(
        functools.partial(_xr_body, nb),
        grid=(nb + 1,),
        in_specs=[
            pl.BlockSpec((bn, D), lambda j, npb=npb: (j % npb, 0)),
            pl.BlockSpec((1, D, D),
                         lambda j, npb=npb, R=R: (jnp.minimum(j // npb,
                                                              R - 1), 0, 0)),
        ],
        out_specs=pl.BlockSpec((bn, D), lambda j: (j, 0)),
        out_shape=jax.ShapeDtypeStruct((R * N + bn, D), jnp.float32),
    )(x, w_l)


def _reduce_body(m_ref, dst_ref, out_ref):
    t = pl.program_id(0)
    k = pl.program_id(1)
    cb = m_ref.shape[0]
    local = dst_ref[...] - t * RANGE
    oh = (local == lax.broadcasted_iota(jnp.int32, (cb, RANGE), 1)
          ).astype(jnp.float32)
    part = lax.dot_general(oh, m_ref[...], (((0,), (0,)), ((), ())),
                           preferred_element_type=jnp.float32)

    @pl.when(k == 0)
    def _():
        out_ref[...] = part

    @pl.when(k > 0)
    def _():
        out_ref[...] = out_ref[...] + part


def _segment_reduce(m, dst2d, D, cb):
    nch = CL // cb
    return pl.pallas_call(
        _reduce_body,
        grid=(NT, nch),
        in_specs=[
            pl.BlockSpec((cb, D), lambda t, k, nch=nch: (t * nch + k, 0)),
            pl.BlockSpec((cb, 1), lambda t, k, nch=nch: (t * nch + k, 0)),
        ],
        out_specs=pl.BlockSpec((RANGE, D), lambda t, k: (t, 0)),
        out_shape=jax.ShapeDtypeStruct((NT * RANGE, D), jnp.float32),
    )(m, dst2d)


def _comb_body(a_ref, d_ref, x_ref, w_ref, b_ref, out_ref):
    deg = jnp.maximum(d_ref[...], 1.0)
    h = a_ref[...] / deg
    h = h + jnp.dot(x_ref[...], w_ref[...],
                    preferred_element_type=jnp.float32) + b_ref[...]
    out_ref[...] = jnp.maximum(h, 0.0)


def _combine(agg, deg, x, w_l, b_l, bn):
    N, D = x.shape
    grid = (N // bn,)
    return pl.pallas_call(
        _comb_body,
        grid=grid,
        in_specs=[
            pl.BlockSpec((bn, D), lambda i: (i, 0)),
            pl.BlockSpec((bn, 1), lambda i: (i, 0)),
            pl.BlockSpec((bn, D), lambda i: (i, 0)),
            pl.BlockSpec((D, D), lambda i: (0, 0)),
            pl.BlockSpec((1, D), lambda i: (0, 0)),
        ],
        out_specs=pl.BlockSpec((bn, D), lambda i: (i, 0)),
        out_shape=jax.ShapeDtypeStruct((N, D), jnp.float32),
    )(agg, deg, x, w_l, b_l)


def _score_body(zh_ref, zt_ref, mp_ref, rel_ref,
                w1_ref, b1_ref, w2_ref, b2_ref, g_ref, bb_ref,
                wga_ref, wgb_ref, bg_ref, remb_ref, out_ref):
    f32 = jnp.float32
    h1 = jnp.maximum(
        jnp.dot(mp_ref[...], w1_ref[...], preferred_element_type=f32)
        + b1_ref[...], 0.0)
    h2 = jnp.dot(h1, w2_ref[...], preferred_element_type=f32) + b2_ref[...]
    mu = jnp.mean(h2, axis=-1, keepdims=True)
    var = jnp.mean((h2 - mu) ** 2, axis=-1, keepdims=True)
    h3 = (h2 - mu) / jnp.sqrt(var + 1e-5) * g_ref[...] + bb_ref[...]
    zm = jnp.maximum(h3, 0.0)
    zt = zt_ref[...]
    logits = (jnp.dot(zt, wga_ref[...], preferred_element_type=f32)
              + jnp.dot(zm, wgb_ref[...], preferred_element_type=f32)
              + bg_ref[...])
    alpha = jax.nn.sigmoid(logits)
    ztf = zt + alpha * zm
    R = remb_ref.shape[0]
    rel = rel_ref[...]
    oh = (rel == lax.broadcasted_iota(jnp.int32, (rel.shape[0], R), 1)
          ).astype(f32)
    r = jnp.dot(oh, remb_ref[...], preferred_element_type=f32)
    out_ref[...] = jnp.sum(zh_ref[...] * r * ztf, axis=-1, keepdims=True)


def _score(zh, zt, mp, rels, W1, b1, W2, b2, ln_g, ln_b, Wga, Wgb, bg,
           relation_emb, bb):
    B, D = zh.shape
    M = mp.shape[1]
    R = relation_emb.shape[0]
    grid = (B // bb,)
    full = lambda shape: pl.BlockSpec(shape, lambda i: tuple(0 for _ in shape))
    return pl.pallas_call(
        _score_body,
        grid=grid,
        in_specs=[
            pl.BlockSpec((bb, D), lambda i: (i, 0)),
            pl.BlockSpec((bb, D), lambda i: (i, 0)),
            pl.BlockSpec((bb, M), lambda i: (i, 0)),
            pl.BlockSpec((bb, 1), lambda i: (i, 0)),
            full((M, D)), full((1, D)), full((D, D)), full((1, D)),
            full((1, D)), full((1, D)), full((D, D)), full((D, D)),
            full((1, D)), full((R, D)),
        ],
        out_specs=pl.BlockSpec((bb, 1), lambda i: (i, 0)),
        out_shape=jax.ShapeDtypeStruct((B, 1), jnp.float32),
    )(zh, zt, mp, rels, W1, b1, W2, b2, ln_g, ln_b, Wga, Wgb, bg,
      relation_emb)


def kernel(node_ids, edge_index, edge_type, heads, rels, tails,
           metapath_feats, entity_emb, W_rel, W_self, b_layer, relation_emb,
           W1, b1, W2, b2, ln_g, ln_b, Wg, bg):
    N, D = entity_emb.shape
    E = edge_index.shape[1]
    L, R = W_rel.shape[0], W_rel.shape[1]
    B = heads.shape[0]

    E_pad = -(-E // W_WIN) * W_WIN
    pad = E_pad - E

    src = edge_index[0]
    dst = edge_index[1]
    if pad:
        pad_i = jnp.arange(pad, dtype=jnp.int32)
        src = jnp.concatenate([src, pad_i % N])
        dst = jnp.concatenate([dst, jnp.full((pad,), NT * RANGE + 1,
                                             jnp.int32)])
        edge_type = jnp.concatenate([edge_type, jnp.zeros((pad,), jnp.int32)])

    prep = _build_prep(N, R, E_pad)
    gl, dl, deg = prep(src, edge_type, dst)
    dl2d = dl.reshape(NT * CL, 1)
    deg2d = deg.reshape(NT * RANGE, 1)

    bn = 400
    gather_m = _build_gather_m(D)

    x = entity_emb
    for l in range(L):
        xr = _relation_transform(x, W_rel[l], bn)
        m = gather_m(xr, gl)
        agg = _segment_reduce(m, dl2d, D, cb=512)
        x = _combine(agg, deg2d, x, W_self[l], b_layer[l].reshape(1, D), bn)

    pair = _build_pair_gather(N, D, B)
    zh, zt = pair(x, heads, tails)

    score = _score(zh, zt, metapath_feats, rels.reshape(B, 1),
                   W1, b1.reshape(1, D), W2, b2.reshape(1, D),
                   ln_g.reshape(1, D), ln_b.reshape(1, D),
                   Wg[:D], Wg[D:], bg.reshape(1, D), relation_emb, bb=512)
    return score.reshape(B)
